# Initial kernel scaffold; baseline (speedup 1.0000x reference)
import jax
import jax.numpy as jnp
from jax.experimental import pallas as pl


def kernel(x, edge_index, edge_attr, A, W1, b1, W2, b2, Wm1, bm1, Wm2, bm2):
    raise NotImplementedError("write your pallas kernel here")


# speedup vs baseline: 7.1183x; 7.1183x over previous
import functools

import jax
import jax.numpy as jnp
from jax import lax
from jax.experimental import pallas as pl
from jax.experimental.pallas import tpu as pltpu
from jax.experimental.pallas import tpu_sc as plsc

N = 10000
E = 320000
D = 128
K = 10

NC = 2
NS = 16
NW = NC * NS
EPW = E // NW
EB = 80
NB = EPW // EB
GNB = E // EB
NPAD = 10240
RPT = NPAD // NS
RB = 1000
LANES = 16

_mesh = plsc.VectorSubcoreMesh(core_axis_name="c", subcore_axis_name="s")


@functools.partial(
    pl.kernel,
    mesh=_mesh,
    out_type=jax.ShapeDtypeStruct((NC, N), jnp.float32),
    scratch_types=[
        pltpu.VMEM((EB,), jnp.int32),
        pltpu.VMEM((EB,), jnp.float32),
        pltpu.VMEM_SHARED((N,), jnp.float32),
    ],
)
def _deg_kernel(col_hbm, ew_hbm, zeros_hbm, out_hbm, col_v, ew_v, deg_sh):
    c = lax.axis_index("c")
    s = lax.axis_index("s")
    wid = s * NC + c

    @pl.when(s == 0)
    def _():
        pltpu.sync_copy(zeros_hbm, deg_sh)

    plsc.subcore_barrier()

    def body(it, carry):
        git = wid * NB + it
        pltpu.sync_copy(col_hbm.at[git], col_v)
        pltpu.sync_copy(ew_hbm.at[git], ew_v)
        pltpu.sync_copy(ew_v, deg_sh.at[col_v], add=True)
        return carry

    lax.fori_loop(0, NB, body, 0)
    plsc.subcore_barrier()

    @pl.when(s == 0)
    def _():
        pltpu.sync_copy(deg_sh, out_hbm.at[c])


@functools.partial(
    pl.kernel,
    mesh=_mesh,
    out_type=jax.ShapeDtypeStruct((NC, NPAD, D), jnp.float32),
    scratch_types=[
        pltpu.VMEM((2, EB), jnp.int32),
        pltpu.VMEM((EB, LANES), jnp.float32),
        pltpu.VMEM((EB, D), jnp.float32),
        pltpu.VMEM_SHARED((NPAD, D), jnp.float32),
        pltpu.SemaphoreType.DMA,
    ],
)
def _edge_kernel(g_hbm, idx_hbm, ew16_hbm, zrows_hbm, out_hbm,
                 idx_v, ew16_v, rows_v, acc_sh, sem):
    c = lax.axis_index("c")
    s = lax.axis_index("s")
    wid = s * NC + c

    pltpu.sync_copy(zrows_hbm, acc_sh.at[pl.ds(s * RPT, RPT)])
    plsc.subcore_barrier()

    def body(it, carry):
        git = wid * NB + it
        pltpu.sync_copy(idx_hbm.at[git], idx_v)
        pltpu.sync_copy(ew16_hbm.at[git], ew16_v)
        pltpu.async_copy(g_hbm.at[idx_v.at[0]], rows_v, sem).wait()

        def scale(e, c2):
            w = ew16_v[e]
            for j in range(D // LANES):
                sl = pl.ds(j * LANES, LANES)
                rows_v[e, sl] = rows_v[e, sl] * w
            return c2

        lax.fori_loop(0, EB, scale, 0)
        pltpu.sync_copy(rows_v, acc_sh.at[idx_v.at[1]], add=True)
        return carry

    lax.fori_loop(0, NB, body, 0)
    plsc.subcore_barrier()
    pltpu.sync_copy(acc_sh.at[pl.ds(s * RPT, RPT)],
                    out_hbm.at[c, pl.ds(s * RPT, RPT)])


def _stage1_body(dp_ref, x_ref, w1_ref, g_ref, dis_ref):
    dp = dp_ref[...]
    deg = dp[0] + dp[1] + 1.0
    dis = lax.rsqrt(deg)
    hw = jnp.dot(x_ref[...], w1_ref[...], preferred_element_type=jnp.float32)
    g_ref[...] = hw * dis
    dis_ref[...] = dis


def _stage1(deg_parts, x, W1):
    return pl.pallas_call(
        _stage1_body,
        grid=(N // RB,),
        in_specs=[
            pl.BlockSpec((NC, RB, 1), lambda i: (0, i, 0)),
            pl.BlockSpec((RB, D), lambda i: (i, 0)),
            pl.BlockSpec((D, D), lambda i: (0, 0)),
        ],
        out_specs=[
            pl.BlockSpec((RB, D), lambda i: (i, 0)),
            pl.BlockSpec((RB, 1), lambda i: (i, 0)),
        ],
        out_shape=[
            jax.ShapeDtypeStruct((N, D), jnp.float32),
            jax.ShapeDtypeStruct((N, 1), jnp.float32),
        ],
    )(deg_parts, x, W1)


def _stage2_body(acc_ref, g_ref, dis_ref, b1_ref, w2_ref, g2_ref):
    acc = acc_ref[...]
    tot = acc[0] + acc[1] + g_ref[...]
    h = jnp.maximum(tot * dis_ref[...] + b1_ref[...], 0.0)
    hw = jnp.dot(h, w2_ref[...], preferred_element_type=jnp.float32)
    g2_ref[...] = hw * dis_ref[...]


def _stage2(acc1, g1, dis, b1, W2):
    return pl.pallas_call(
        _stage2_body,
        grid=(N // RB,),
        in_specs=[
            pl.BlockSpec((NC, RB, D), lambda i: (0, i, 0)),
            pl.BlockSpec((RB, D), lambda i: (i, 0)),
            pl.BlockSpec((RB, 1), lambda i: (i, 0)),
            pl.BlockSpec((D,), lambda i: (0,)),
            pl.BlockSpec((D, D), lambda i: (0, 0)),
        ],
        out_specs=pl.BlockSpec((RB, D), lambda i: (i, 0)),
        out_shape=jax.ShapeDtypeStruct((N, D), jnp.float32),
    )(acc1, g1, dis, b1, W2)


def _elu(v):
    return jnp.where(v > 0.0, v, jnp.exp(v) - 1.0)


def _stage3_body(acc_ref, g_ref, dis_ref, b2_ref, wm1_ref, bm1_ref,
                 wm2_ref, bm2_ref, s_ref):
    acc = acc_ref[...]
    tot = acc[0] + acc[1] + g_ref[...]
    h = _elu(tot * dis_ref[...] + b2_ref[...])
    t = _elu(jnp.dot(h, wm1_ref[...], preferred_element_type=jnp.float32)
             + bm1_ref[...])
    hcl = jnp.dot(t, wm2_ref[...], preferred_element_type=jnp.float32) + bm2_ref[...]
    m = jnp.max(hcl, axis=-1, keepdims=True)
    ex = jnp.exp(hcl - m)
    s_ref[...] = ex / jnp.sum(ex, axis=-1, keepdims=True)


def _stage3(acc2, g2, dis, b2, Wm1, bm1, Wm2, bm2):
    return pl.pallas_call(
        _stage3_body,
        grid=(N // RB,),
        in_specs=[
            pl.BlockSpec((NC, RB, D), lambda i: (0, i, 0)),
            pl.BlockSpec((RB, D), lambda i: (i, 0)),
            pl.BlockSpec((RB, 1), lambda i: (i, 0)),
            pl.BlockSpec((D,), lambda i: (0,)),
            pl.BlockSpec((D, D), lambda i: (0, 0)),
            pl.BlockSpec((D,), lambda i: (0,)),
            pl.BlockSpec((D, K), lambda i: (0, 0)),
            pl.BlockSpec((K,), lambda i: (0,)),
        ],
        out_specs=pl.BlockSpec((RB, K), lambda i: (i, 0)),
        out_shape=jax.ShapeDtypeStruct((N, K), jnp.float32),
    )(acc2, g2, dis, b2, Wm1, bm1, Wm2, bm2)


def kernel(x, edge_index, edge_attr, A, W1, b1, W2, b2, Wm1, bm1, Wm2, bm2):
    row = edge_index[0].astype(jnp.int32).reshape(GNB, EB)
    col = edge_index[1].astype(jnp.int32).reshape(GNB, EB)
    idx = jnp.stack([row, col], axis=1)
    ew16 = jnp.broadcast_to(edge_attr.reshape(GNB, EB, 1), (GNB, EB, LANES))

    zeros_n = jnp.zeros((N,), jnp.float32)
    zeros_rows = jnp.zeros((RPT, D), jnp.float32)

    deg_parts = _deg_kernel(col, edge_attr.reshape(GNB, EB), zeros_n)
    g1, dis = _stage1(deg_parts.reshape(NC, N, 1), x, W1)
    acc1 = _edge_kernel(g1, idx, ew16, zeros_rows)
    g2 = _stage2(acc1, g1, dis, b1, W2)
    acc2 = _edge_kernel(g2, idx, ew16, zeros_rows)
    S = _stage3(acc2, g2, dis, b2, Wm1, bm1, Wm2, bm2)
    return (A, S)
